# baseline (speedup 1.0000x reference)
import functools

import jax
import jax.numpy as jnp
from jax import lax
from jax.experimental import pallas as pl
from jax.experimental.pallas import tpu as pltpu
from jax.experimental.pallas import tpu_sc as plsc

N = 10000
E = 160000
C = 128
NPAD = 10240
EPAD = 163840
NC, NS = 2, 16
NW = NC * NS
EPW = EPAD // NW
CH = 128
GCH = EPW // CH
EPS = EPAD // NS
SCH = EPS // CH
RPS = NPAD // NS

BRN = 512
BRE = 1280

_f32 = jnp.float32


@functools.lru_cache(maxsize=None)
def _sc_kernels():
    mesh = plsc.VectorSubcoreMesh(
        core_axis_name="c", subcore_axis_name="s",
        num_cores=NC, num_subcores=NS)

    NB = 5

    CG = 64
    GCHG = EPW // CG
    NBG = 4

    @functools.partial(
        pl.kernel,
        out_type=[jax.ShapeDtypeStruct((EPAD, C), _f32),
                  jax.ShapeDtypeStruct((EPAD, C), _f32)],
        mesh=mesh,
        scratch_types=[pltpu.VMEM((2, EPW), jnp.int32),
                       pltpu.VMEM((NBG, CG, C), _f32),
                       pltpu.VMEM_SHARED((NPAD, C), _f32),
                       pltpu.SemaphoreType.DMA,
                       pltpu.SemaphoreType.DMA],
    )
    def sc_gather(y_hbm, ij_hbm, gi_hbm, gj_hbm, idx_v, rows_v, y_sh,
                  gsem, osem):
        c = lax.axis_index("c")
        s = lax.axis_index("s")
        wid = s * NC + c
        ebase = wid * EPW
        for k in range(RPS // CG):
            rbase = s * RPS + k * CG
            pltpu.sync_copy(y_hbm.at[pl.ds(rbase, CG)], rows_v.at[0])
            pltpu.sync_copy(rows_v.at[0], y_sh.at[pl.ds(rbase, CG)])
        for a in (0, 1):
            pltpu.sync_copy(ij_hbm.at[a, pl.ds(ebase, EPW)], idx_v.at[a])
        plsc.subcore_barrier()
        for a, out_hbm in ((0, gi_hbm), (1, gj_hbm)):
            def body(k, carry, a=a, out_hbm=out_hbm):
                gs, os = [], []
                for b in range(NBG):
                    off = pl.multiple_of((k * NBG + b) * CG, CG)
                    gs.append(pltpu.async_copy(
                        y_sh.at[idx_v.at[a, pl.ds(off, CG)]],
                        rows_v.at[b], gsem))
                for b in range(NBG):
                    gs[b].wait()
                    off = pl.multiple_of((k * NBG + b) * CG, CG)
                    os.append(pltpu.async_copy(
                        rows_v.at[b], out_hbm.at[pl.ds(ebase + off, CG)],
                        osem))
                for o in os:
                    o.wait()
                return carry
            lax.fori_loop(0, GCHG // NBG, body, 0)

    CS = 64
    SCHS = EPS // CS
    NBS = 4

    @functools.partial(
        pl.kernel,
        out_type=jax.ShapeDtypeStruct((2, NPAD, C), _f32),
        mesh=mesh,
        scratch_types=[pltpu.VMEM((SCHS // 2, CS), jnp.int32),
                       pltpu.VMEM((NBS, CS, C), _f32),
                       pltpu.VMEM_SHARED((NPAD, C), _f32),
                       pltpu.SemaphoreType.DMA,
                       pltpu.SemaphoreType.DMA],
    )
    def sc_scatter(xe_hbm, ij5_hbm, s_hbm, idx_v, rows_v, acc_sh,
                   lsem, ssem):
        c = lax.axis_index("c")
        s = lax.axis_index("s")
        zv = jnp.zeros((16,), _f32)

        def zbody(r, carry):
            for k in range(8):
                rows_v[0, r, pl.ds(k * 16, 16)] = zv
            return carry

        lax.fori_loop(0, CS, zbody, 0)
        for k in range(RPS // CS):
            pltpu.sync_copy(rows_v.at[0],
                            acc_sh.at[pl.ds(s * RPS + k * CS, CS)])
        plsc.subcore_barrier()

        for h in range(2):
            pltpu.sync_copy(ij5_hbm.at[c, s, h], idx_v)

            def body(k, carry, h=h):
                ls, ss = [], []
                for b in range(NBS):
                    t = k * NBS + b
                    base = pl.multiple_of(
                        s * EPS + (h * (SCHS // 2) + t) * CS, CS)
                    ls.append(pltpu.async_copy(
                        xe_hbm.at[pl.ds(base, CS)], rows_v.at[b], lsem))
                for b in range(NBS):
                    ls[b].wait()
                    t = k * NBS + b
                    ss.append(pltpu.async_copy(
                        rows_v.at[b], acc_sh.at[idx_v.at[t]], ssem, add=True))
                for x in ss:
                    x.wait()
                return carry

            lax.fori_loop(0, SCHS // 2 // NBS, body, 0)
        plsc.subcore_barrier()
        pltpu.sync_copy(acc_sh.at[pl.ds(s * RPS, RPS)],
                        s_hbm.at[c, pl.ds(s * RPS, RPS)])

    return sc_gather, sc_scatter


def _dot(a, b):
    return jnp.dot(a, b, preferred_element_type=_f32)


def _a1_body(xn_ref, kno_ref, kn_ref, kna_ref, xn0_ref, y_ref):
    xn0 = _dot(xn_ref[...], kno_ref[...])
    xn0_ref[...] = xn0
    wg = jnp.concatenate([kn_ref[...], 0.5 * kna_ref[...]], axis=1)
    y_ref[...] = _dot(xn0, wg)


def _tc_open_node(xnT, knoT, knT, knaT):
    return pl.pallas_call(
        _a1_body,
        grid=(NPAD // BRN,),
        in_specs=[pl.BlockSpec((BRN, C), lambda i: (i, 0)),
                  pl.BlockSpec((C, C), lambda i: (0, 0)),
                  pl.BlockSpec((C, 64), lambda i: (0, 0)),
                  pl.BlockSpec((C, 64), lambda i: (0, 0))],
        out_specs=[pl.BlockSpec((BRN, C), lambda i: (i, 0)),
                   pl.BlockSpec((BRN, C), lambda i: (i, 0))],
        out_shape=[jax.ShapeDtypeStruct((NPAD, C), _f32),
                   jax.ShapeDtypeStruct((NPAD, C), _f32)],
    )(xnT, knoT, knT, knaT)


def _a2_body(xe_ref, keo_ref, out_ref):
    out_ref[...] = _dot(xe_ref[...], keo_ref[...])


def _tc_open_edge(xeT, keoT):
    return pl.pallas_call(
        _a2_body,
        grid=(EPAD // BRE,),
        in_specs=[pl.BlockSpec((BRE, 16), lambda i: (i, 0)),
                  pl.BlockSpec((16, C), lambda i: (0, 0))],
        out_specs=pl.BlockSpec((BRE, C), lambda i: (i, 0)),
        out_shape=jax.ShapeDtypeStruct((EPAD, C), _f32),
    )(xeT, keoT)


def _edge_update(gi, gj, xe):
    ch = lax.broadcasted_iota(jnp.int32, gi.shape, 1)
    a = jnp.where(ch < 64, gi - gj, gi + gj)
    a = a - jnp.mean(a, axis=1, keepdims=True)
    a = a / jnp.sqrt(jnp.sum(a * a, axis=1, keepdims=True) + 1e-3)
    return xe + 0.1 * jnp.maximum(a, 0.0)


def _e_body(gi_ref, gj_ref, xet_ref, keo_ref, out_ref):
    xe0 = _dot(xet_ref[...], keo_ref[...])
    xe_new = _edge_update(gi_ref[...], gj_ref[...], xe0)
    valid = pl.program_id(0) < (E // BRE)
    out_ref[...] = jnp.where(valid, xe_new, 0.0)


def _tc_edge_open(gi, gj, xeT, keoT):
    return pl.pallas_call(
        _e_body,
        grid=(EPAD // BRE,),
        in_specs=[pl.BlockSpec((BRE, C), lambda i: (i, 0))] * 2 +
                 [pl.BlockSpec((BRE, 16), lambda i: (i, 0)),
                  pl.BlockSpec((16, C), lambda i: (0, 0))],
        out_specs=pl.BlockSpec((BRE, C), lambda i: (i, 0)),
        out_shape=jax.ShapeDtypeStruct((EPAD, C), _f32),
    )(gi, gj, xeT, keoT)


def _e2_body(gi_ref, gj_ref, xe_ref, out_ref):
    xe_new = _edge_update(gi_ref[...], gj_ref[...], xe_ref[...])
    valid = pl.program_id(0) < (E // BRE)
    out_ref[...] = jnp.where(valid, xe_new, 0.0)


def _tc_edge2(gi, gj, xe):
    return pl.pallas_call(
        _e2_body,
        grid=(EPAD // BRE,),
        in_specs=[pl.BlockSpec((BRE, C), lambda i: (i, 0))] * 3,
        out_specs=pl.BlockSpec((BRE, C), lambda i: (i, 0)),
        out_shape=jax.ShapeDtypeStruct((EPAD, C), _f32),
    )(gi, gj, xe)


def _cl_body(xe_ref, kec_ref, cl_ref):
    cl_ref[...] = lax.dot_general(kec_ref[...], xe_ref[...],
                                  (((1,), (1,)), ((), ())),
                                  preferred_element_type=_f32)[None, :, :, None]


def _tc_close_edge(xe, kec):
    return pl.pallas_call(
        _cl_body,
        grid=(E // BRE,),
        in_specs=[pl.BlockSpec((BRE, C), lambda i: (i, 0)),
                  pl.BlockSpec((16, C), lambda i: (0, 0))],
        out_specs=pl.BlockSpec((1, 16, BRE, 1), lambda i: (0, 0, i, 0)),
        out_shape=jax.ShapeDtypeStruct((1, 16, E, 1), _f32),
    )(xe, kec)


def _node_new(xn_ref, si_ref, sj_ref, kei_ref, kea_ref):
    wi = 0.1 * (kei_ref[...] + 0.5 * kea_ref[...])
    wj = 0.1 * (0.5 * kea_ref[...] - kei_ref[...])
    return xn_ref[...] + _dot(si_ref[0], wi) + _dot(sj_ref[0], wj)


def _n_body(xn_ref, si_ref, sj_ref, kei_ref, kea_ref, kn_ref, kna_ref,
            xn1_ref, y_ref):
    xn1 = _node_new(xn_ref, si_ref, sj_ref, kei_ref, kea_ref)
    xn1_ref[...] = xn1
    wg = jnp.concatenate([kn_ref[...], 0.5 * kna_ref[...]], axis=1)
    y_ref[...] = _dot(xn1, wg)


def _tc_node(xn, s2, keiT, keaT, knT, knaT):
    return pl.pallas_call(
        _n_body,
        grid=(NPAD // BRN,),
        in_specs=[pl.BlockSpec((BRN, C), lambda i: (i, 0)),
                  pl.BlockSpec((1, BRN, C), lambda i: (0, i, 0)),
                  pl.BlockSpec((1, BRN, C), lambda i: (1, i, 0)),
                  pl.BlockSpec((C, C), lambda i: (0, 0)),
                  pl.BlockSpec((C, C), lambda i: (0, 0)),
                  pl.BlockSpec((C, 64), lambda i: (0, 0)),
                  pl.BlockSpec((C, 64), lambda i: (0, 0))],
        out_specs=[pl.BlockSpec((BRN, C), lambda i: (i, 0)),
                   pl.BlockSpec((BRN, C), lambda i: (i, 0))],
        out_shape=[jax.ShapeDtypeStruct((NPAD, C), _f32),
                   jax.ShapeDtypeStruct((NPAD, C), _f32)],
    )(xn, s2, s2, keiT, keaT, knT, knaT)


def _nf_body(xn_ref, si_ref, sj_ref, kei_ref, kea_ref, knc_ref, out_ref):
    xn1 = _node_new(xn_ref, si_ref, sj_ref, kei_ref, kea_ref)
    out_ref[...] = lax.dot_general(knc_ref[...], xn1,
                                   (((1,), (1,)), ((), ())),
                                   preferred_element_type=_f32)[None]


def _tc_node_final(xn, s2, keiT, keaT, knc):
    return pl.pallas_call(
        _nf_body,
        grid=(NPAD // BRN,),
        in_specs=[pl.BlockSpec((BRN, C), lambda i: (i, 0)),
                  pl.BlockSpec((1, BRN, C), lambda i: (0, i, 0)),
                  pl.BlockSpec((1, BRN, C), lambda i: (1, i, 0)),
                  pl.BlockSpec((C, C), lambda i: (0, 0)),
                  pl.BlockSpec((C, C), lambda i: (0, 0)),
                  pl.BlockSpec((C, C), lambda i: (0, 0))],
        out_specs=pl.BlockSpec((1, C, BRN), lambda i: (0, 0, i)),
        out_shape=jax.ShapeDtypeStruct((1, C, N), _f32),
    )(xn, s2, s2, keiT, keaT, knc)


def kernel(xn, xe, edge_index, KNopen, KEopen, KNclose, KEclose,
           KN, KE, KNa, KEa):
    xnT = jnp.pad(xn[0].T, ((0, NPAD - N), (0, 0)))
    xeT = jnp.pad(xe[0, :, :, 0].T, ((0, EPAD - E), (0, 0)))
    ij = jnp.pad(edge_index, ((0, 0), (0, EPAD - E)))
    ij4 = ij.reshape(2, NS, 2, EPS // 128, 64)

    _sc_gather, _sc_scatter = _sc_kernels()

    knT = [KN[l].T for l in range(2)]
    knaT = [KNa[l].T for l in range(2)]
    keT = [KE[l].T for l in range(2)]
    keaT = [KEa[l].T for l in range(2)]

    xn0, y = _tc_open_node(xnT, KNopen.T, knT[0], knaT[0])

    gi, gj = _sc_gather(y, ij)
    xe1 = _tc_edge_open(gi, gj, xeT, KEopen.T)
    s2 = _sc_scatter(xe1, ij4)
    xn1, y = _tc_node(xn0, s2, keT[0], keaT[0], knT[1], knaT[1])

    gi, gj = _sc_gather(y, ij)
    xe2 = _tc_edge2(gi, gj, xe1)
    s2 = _sc_scatter(xe2, ij4)
    xecl = _tc_close_edge(xe2, KEclose)
    xncl = _tc_node_final(xn1, s2, keT[1], keaT[1], KNclose)

    return (xncl, xecl)


# speedup vs baseline: 2.1445x; 2.1445x over previous
import functools

import jax
import jax.numpy as jnp
from jax import lax
from jax.experimental import pallas as pl
from jax.experimental.pallas import tpu as pltpu
from jax.experimental.pallas import tpu_sc as plsc

N = 10000
E = 160000
C = 128
NPAD = 10240
EPAD = 163840
NC, NS = 2, 16
NW = NC * NS
EPW = EPAD // NW
CH = 128
GCH = EPW // CH
EPS = EPAD // NS
SCH = EPS // CH
RPS = NPAD // NS

BRN = 512
BRE = 1280

_f32 = jnp.float32


@functools.lru_cache(maxsize=None)
def _sc_kernels():
    mesh = plsc.VectorSubcoreMesh(
        core_axis_name="c", subcore_axis_name="s",
        num_cores=NC, num_subcores=NS)

    NB = 5

    CG = 64
    GCHG = EPW // CG
    NBG = 4

    @functools.partial(
        pl.kernel,
        out_type=[jax.ShapeDtypeStruct((EPAD, C), _f32),
                  jax.ShapeDtypeStruct((EPAD, C), _f32)],
        mesh=mesh,
        scratch_types=[pltpu.VMEM((2, EPW), jnp.int32),
                       pltpu.VMEM((NBG, CG, C), _f32),
                       pltpu.VMEM_SHARED((NPAD, C), _f32),
                       pltpu.SemaphoreType.DMA,
                       pltpu.SemaphoreType.DMA],
    )
    def sc_gather(y_hbm, ij_hbm, gi_hbm, gj_hbm, idx_v, rows_v, y_sh,
                  gsem, osem):
        c = lax.axis_index("c")
        s = lax.axis_index("s")
        wid = s * NC + c
        ebase = wid * EPW
        for k in range(RPS // CG):
            rbase = s * RPS + k * CG
            pltpu.sync_copy(y_hbm.at[pl.ds(rbase, CG)], rows_v.at[0])
            pltpu.sync_copy(rows_v.at[0], y_sh.at[pl.ds(rbase, CG)])
        for a in (0, 1):
            pltpu.sync_copy(ij_hbm.at[a, pl.ds(ebase, EPW)], idx_v.at[a])
        plsc.subcore_barrier()
        for a, out_hbm in ((0, gi_hbm), (1, gj_hbm)):
            def body(k, carry, a=a, out_hbm=out_hbm):
                gs, os = [], []
                for b in range(NBG):
                    off = pl.multiple_of((k * NBG + b) * CG, CG)
                    gs.append(pltpu.async_copy(
                        y_sh.at[idx_v.at[a, pl.ds(off, CG)]],
                        rows_v.at[b], gsem))
                for b in range(NBG):
                    gs[b].wait()
                    off = pl.multiple_of((k * NBG + b) * CG, CG)
                    os.append(pltpu.async_copy(
                        rows_v.at[b], out_hbm.at[pl.ds(ebase + off, CG)],
                        osem))
                for o in os:
                    o.wait()
                return carry
            lax.fori_loop(0, GCHG // NBG, body, 0)

    CS = 64
    SCHS = EPS // CS
    NBS = 4

    @functools.partial(
        pl.kernel,
        out_type=jax.ShapeDtypeStruct((2, NPAD, C), _f32),
        mesh=mesh,
        scratch_types=[pltpu.VMEM((SCHS // 2, CS), jnp.int32),
                       pltpu.VMEM((NBS, CS, C), _f32),
                       pltpu.VMEM_SHARED((NPAD, C), _f32),
                       pltpu.SemaphoreType.DMA,
                       pltpu.SemaphoreType.DMA],
    )
    def sc_scatter(xe_hbm, ij5_hbm, s_hbm, idx_v, rows_v, acc_sh,
                   lsem, ssem):
        c = lax.axis_index("c")
        s = lax.axis_index("s")
        zv = jnp.zeros((16,), _f32)

        def zbody(r, carry):
            for k in range(8):
                rows_v[0, r, pl.ds(k * 16, 16)] = zv
            return carry

        lax.fori_loop(0, CS, zbody, 0)
        for k in range(RPS // CS):
            pltpu.sync_copy(rows_v.at[0],
                            acc_sh.at[pl.ds(s * RPS + k * CS, CS)])
        plsc.subcore_barrier()

        for h in range(2):
            pltpu.sync_copy(ij5_hbm.at[c, s, h], idx_v)

            def body(k, carry, h=h):
                ls, ss = [], []
                for b in range(NBS):
                    t = k * NBS + b
                    base = pl.multiple_of(
                        s * EPS + (h * (SCHS // 2) + t) * CS, CS)
                    ls.append(pltpu.async_copy(
                        xe_hbm.at[pl.ds(base, CS)], rows_v.at[b], lsem))
                for b in range(NBS):
                    ls[b].wait()
                    t = k * NBS + b
                    ss.append(pltpu.async_copy(
                        rows_v.at[b], acc_sh.at[idx_v.at[t]], ssem, add=True))
                for x in ss:
                    x.wait()
                return carry

            lax.fori_loop(0, SCHS // 2 // NBS, body, 0)
        plsc.subcore_barrier()
        pltpu.sync_copy(acc_sh.at[pl.ds(s * RPS, RPS)],
                        s_hbm.at[c, pl.ds(s * RPS, RPS)])

    return sc_gather, sc_scatter


def _dot(a, b):
    return jnp.dot(a, b, preferred_element_type=_f32)


def _a1_body(xn_ref, kno_ref, kn_ref, kna_ref, xn0_ref, y_ref):
    xn0 = _dot(xn_ref[...], kno_ref[...])
    xn0_ref[...] = xn0
    wg = jnp.concatenate([kn_ref[...], 0.5 * kna_ref[...]], axis=1)
    y_ref[...] = _dot(xn0, wg)


def _tc_open_node(xnT, knoT, knT, knaT):
    return pl.pallas_call(
        _a1_body,
        grid=(NPAD // BRN,),
        in_specs=[pl.BlockSpec((BRN, C), lambda i: (i, 0)),
                  pl.BlockSpec((C, C), lambda i: (0, 0)),
                  pl.BlockSpec((C, 64), lambda i: (0, 0)),
                  pl.BlockSpec((C, 64), lambda i: (0, 0))],
        out_specs=[pl.BlockSpec((BRN, C), lambda i: (i, 0)),
                   pl.BlockSpec((BRN, C), lambda i: (i, 0))],
        out_shape=[jax.ShapeDtypeStruct((NPAD, C), _f32),
                   jax.ShapeDtypeStruct((NPAD, C), _f32)],
    )(xnT, knoT, knT, knaT)


def _a2_body(xe_ref, keo_ref, out_ref):
    out_ref[...] = _dot(xe_ref[...], keo_ref[...])


def _tc_open_edge(xeT, keoT):
    return pl.pallas_call(
        _a2_body,
        grid=(EPAD // BRE,),
        in_specs=[pl.BlockSpec((BRE, 16), lambda i: (i, 0)),
                  pl.BlockSpec((16, C), lambda i: (0, 0))],
        out_specs=pl.BlockSpec((BRE, C), lambda i: (i, 0)),
        out_shape=jax.ShapeDtypeStruct((EPAD, C), _f32),
    )(xeT, keoT)


def _edge_update(gi, gj, xe):
    ch = lax.broadcasted_iota(jnp.int32, gi.shape, 1)
    a = jnp.where(ch < 64, gi - gj, gi + gj)
    a = a - jnp.mean(a, axis=1, keepdims=True)
    a = a / jnp.sqrt(jnp.sum(a * a, axis=1, keepdims=True) + 1e-3)
    return xe + 0.1 * jnp.maximum(a, 0.0)


def _e_body(gi_ref, gj_ref, xet_ref, keo_ref, out_ref):
    xe0 = _dot(xet_ref[...], keo_ref[...])
    xe_new = _edge_update(gi_ref[...], gj_ref[...], xe0)
    valid = pl.program_id(0) < (E // BRE)
    out_ref[...] = jnp.where(valid, xe_new, 0.0)


def _tc_edge_open(gi, gj, xeT, keoT):
    return pl.pallas_call(
        _e_body,
        grid=(EPAD // BRE,),
        in_specs=[pl.BlockSpec((BRE, C), lambda i: (i, 0))] * 2 +
                 [pl.BlockSpec((BRE, 16), lambda i: (i, 0)),
                  pl.BlockSpec((16, C), lambda i: (0, 0))],
        out_specs=pl.BlockSpec((BRE, C), lambda i: (i, 0)),
        out_shape=jax.ShapeDtypeStruct((EPAD, C), _f32),
    )(gi, gj, xeT, keoT)


def _ec_body(gi_ref, gj_ref, xe_ref, kec_ref, out_ref, cl_ref):
    xe_new = _edge_update(gi_ref[...], gj_ref[...], xe_ref[...])
    valid = pl.program_id(0) < (E // BRE)
    xe_new = jnp.where(valid, xe_new, 0.0)
    out_ref[...] = xe_new
    cl_ref[...] = lax.dot_general(kec_ref[...], xe_new,
                                  (((1,), (1,)), ((), ())),
                                  preferred_element_type=_f32)


def _tc_edge_close(gi, gj, xe, kec):
    return pl.pallas_call(
        _ec_body,
        grid=(EPAD // BRE,),
        in_specs=[pl.BlockSpec((BRE, C), lambda i: (i, 0))] * 3 +
                 [pl.BlockSpec((16, C), lambda i: (0, 0))],
        out_specs=[pl.BlockSpec((BRE, C), lambda i: (i, 0)),
                   pl.BlockSpec((16, BRE), lambda i: (0, i))],
        out_shape=[jax.ShapeDtypeStruct((EPAD, C), _f32),
                   jax.ShapeDtypeStruct((16, EPAD), _f32)],
    )(gi, gj, xe, kec)


def _node_new(xn_ref, si_ref, sj_ref, kei_ref, kea_ref):
    wi = 0.1 * (kei_ref[...] + 0.5 * kea_ref[...])
    wj = 0.1 * (0.5 * kea_ref[...] - kei_ref[...])
    return xn_ref[...] + _dot(si_ref[0], wi) + _dot(sj_ref[0], wj)


def _n_body(xn_ref, si_ref, sj_ref, kei_ref, kea_ref, kn_ref, kna_ref,
            xn1_ref, y_ref):
    xn1 = _node_new(xn_ref, si_ref, sj_ref, kei_ref, kea_ref)
    xn1_ref[...] = xn1
    wg = jnp.concatenate([kn_ref[...], 0.5 * kna_ref[...]], axis=1)
    y_ref[...] = _dot(xn1, wg)


def _tc_node(xn, s2, keiT, keaT, knT, knaT):
    return pl.pallas_call(
        _n_body,
        grid=(NPAD // BRN,),
        in_specs=[pl.BlockSpec((BRN, C), lambda i: (i, 0)),
                  pl.BlockSpec((1, BRN, C), lambda i: (0, i, 0)),
                  pl.BlockSpec((1, BRN, C), lambda i: (1, i, 0)),
                  pl.BlockSpec((C, C), lambda i: (0, 0)),
                  pl.BlockSpec((C, C), lambda i: (0, 0)),
                  pl.BlockSpec((C, 64), lambda i: (0, 0)),
                  pl.BlockSpec((C, 64), lambda i: (0, 0))],
        out_specs=[pl.BlockSpec((BRN, C), lambda i: (i, 0)),
                   pl.BlockSpec((BRN, C), lambda i: (i, 0))],
        out_shape=[jax.ShapeDtypeStruct((NPAD, C), _f32),
                   jax.ShapeDtypeStruct((NPAD, C), _f32)],
    )(xn, s2, s2, keiT, keaT, knT, knaT)


def _nf_body(xn_ref, si_ref, sj_ref, kei_ref, kea_ref, knc_ref, out_ref):
    xn1 = _node_new(xn_ref, si_ref, sj_ref, kei_ref, kea_ref)
    out_ref[...] = lax.dot_general(knc_ref[...], xn1,
                                   (((1,), (1,)), ((), ())),
                                   preferred_element_type=_f32)


def _tc_node_final(xn, s2, keiT, keaT, knc):
    return pl.pallas_call(
        _nf_body,
        grid=(NPAD // BRN,),
        in_specs=[pl.BlockSpec((BRN, C), lambda i: (i, 0)),
                  pl.BlockSpec((1, BRN, C), lambda i: (0, i, 0)),
                  pl.BlockSpec((1, BRN, C), lambda i: (1, i, 0)),
                  pl.BlockSpec((C, C), lambda i: (0, 0)),
                  pl.BlockSpec((C, C), lambda i: (0, 0)),
                  pl.BlockSpec((C, C), lambda i: (0, 0))],
        out_specs=pl.BlockSpec((C, BRN), lambda i: (0, i)),
        out_shape=jax.ShapeDtypeStruct((C, NPAD), _f32),
    )(xn, s2, s2, keiT, keaT, knc)


def kernel(xn, xe, edge_index, KNopen, KEopen, KNclose, KEclose,
           KN, KE, KNa, KEa):
    xnT = jnp.pad(xn[0].T, ((0, NPAD - N), (0, 0)))
    xeT = jnp.pad(xe[0, :, :, 0].T, ((0, EPAD - E), (0, 0)))
    ij = jnp.pad(edge_index, ((0, 0), (0, EPAD - E)))
    ij4 = ij.reshape(2, NS, 2, EPS // 128, 64)

    _sc_gather, _sc_scatter = _sc_kernels()

    knT = [KN[l].T for l in range(2)]
    knaT = [KNa[l].T for l in range(2)]
    keT = [KE[l].T for l in range(2)]
    keaT = [KEa[l].T for l in range(2)]

    xn0, y = _tc_open_node(xnT, KNopen.T, knT[0], knaT[0])

    gi, gj = _sc_gather(y, ij)
    xe1 = _tc_edge_open(gi, gj, xeT, KEopen.T)
    s2 = _sc_scatter(xe1, ij4)
    xn1, y = _tc_node(xn0, s2, keT[0], keaT[0], knT[1], knaT[1])

    gi, gj = _sc_gather(y, ij)
    xe2, xeclT = _tc_edge_close(gi, gj, xe1, KEclose)
    s2 = _sc_scatter(xe2, ij4)
    xnclT = _tc_node_final(xn1, s2, keT[1], keaT[1], KNclose)

    return (xnclT[None, :, :N], xeclT[:, :E][None, :, :, None])


# speedup vs baseline: 2.1483x; 1.0018x over previous
import functools

import jax
import jax.numpy as jnp
from jax import lax
from jax.experimental import pallas as pl
from jax.experimental.pallas import tpu as pltpu
from jax.experimental.pallas import tpu_sc as plsc

N = 10000
E = 160000
C = 128
NPAD = 10240
EPAD = 163840
NC, NS = 2, 16
NW = NC * NS
EPW = EPAD // NW
CH = 128
GCH = EPW // CH
EPS = EPAD // NS
SCH = EPS // CH
RPS = NPAD // NS

BRN = 512
BRE = 1280

_f32 = jnp.float32


@functools.lru_cache(maxsize=None)
def _sc_kernels():
    mesh = plsc.VectorSubcoreMesh(
        core_axis_name="c", subcore_axis_name="s",
        num_cores=NC, num_subcores=NS)

    NB = 5

    CG = 64
    GCHG = EPW // CG
    NBG = 4

    @functools.partial(
        pl.kernel,
        out_type=[jax.ShapeDtypeStruct((EPAD, C), _f32),
                  jax.ShapeDtypeStruct((EPAD, C), _f32)],
        mesh=mesh,
        scratch_types=[pltpu.VMEM((2, EPW), jnp.int32),
                       pltpu.VMEM((NBG, CG, C), _f32),
                       pltpu.VMEM_SHARED((NPAD, C), _f32),
                       pltpu.SemaphoreType.DMA,
                       pltpu.SemaphoreType.DMA],
    )
    def sc_gather(y_hbm, ij_hbm, gi_hbm, gj_hbm, idx_v, rows_v, y_sh,
                  gsem, osem):
        c = lax.axis_index("c")
        s = lax.axis_index("s")
        wid = s * NC + c
        ebase = wid * EPW
        for k in range(RPS // CG):
            rbase = s * RPS + k * CG
            pltpu.sync_copy(y_hbm.at[pl.ds(rbase, CG)], rows_v.at[0])
            pltpu.sync_copy(rows_v.at[0], y_sh.at[pl.ds(rbase, CG)])
        for a in (0, 1):
            pltpu.sync_copy(ij_hbm.at[a, pl.ds(ebase, EPW)], idx_v.at[a])
        plsc.subcore_barrier()
        for a, out_hbm in ((0, gi_hbm), (1, gj_hbm)):
            def body(k, carry, a=a, out_hbm=out_hbm):
                gs, os = [], []
                for b in range(NBG):
                    off = pl.multiple_of((k * NBG + b) * CG, CG)
                    gs.append(pltpu.async_copy(
                        y_sh.at[idx_v.at[a, pl.ds(off, CG)]],
                        rows_v.at[b], gsem))
                for b in range(NBG):
                    gs[b].wait()
                    off = pl.multiple_of((k * NBG + b) * CG, CG)
                    os.append(pltpu.async_copy(
                        rows_v.at[b], out_hbm.at[pl.ds(ebase + off, CG)],
                        osem))
                for o in os:
                    o.wait()
                return carry
            lax.fori_loop(0, GCHG // NBG, body, 0)

    CS = 64
    SCHS = EPS // CS
    NBS = 4

    @functools.partial(
        pl.kernel,
        out_type=jax.ShapeDtypeStruct((2, NPAD, C), _f32),
        mesh=mesh,
        scratch_types=[pltpu.VMEM((SCHS // 2, CS), jnp.int32),
                       pltpu.VMEM((NBS, CS, C), _f32),
                       pltpu.VMEM_SHARED((NPAD, C), _f32),
                       pltpu.SemaphoreType.DMA,
                       pltpu.SemaphoreType.DMA],
    )
    def sc_scatter(xe_hbm, ij5_hbm, s_hbm, idx_v, rows_v, acc_sh,
                   lsem, ssem):
        c = lax.axis_index("c")
        s = lax.axis_index("s")
        zv = jnp.zeros((16,), _f32)

        def zbody(r, carry):
            for k in range(8):
                rows_v[0, r, pl.ds(k * 16, 16)] = zv
            return carry

        lax.fori_loop(0, CS, zbody, 0)
        for k in range(RPS // CS):
            pltpu.sync_copy(rows_v.at[0],
                            acc_sh.at[pl.ds(s * RPS + k * CS, CS)])
        plsc.subcore_barrier()

        for h in range(2):
            pltpu.sync_copy(ij5_hbm.at[c, s, h], idx_v)

            def body(k, carry, h=h):
                ls, ss = [], []
                for b in range(NBS):
                    t = k * NBS + b
                    base = pl.multiple_of(
                        s * EPS + (h * (SCHS // 2) + t) * CS, CS)
                    ls.append(pltpu.async_copy(
                        xe_hbm.at[pl.ds(base, CS)], rows_v.at[b], lsem))
                for b in range(NBS):
                    ls[b].wait()
                    t = k * NBS + b
                    ss.append(pltpu.async_copy(
                        rows_v.at[b], acc_sh.at[idx_v.at[t]], ssem, add=True))
                for x in ss:
                    x.wait()
                return carry

            lax.fori_loop(0, SCHS // 2 // NBS, body, 0)
        plsc.subcore_barrier()
        pltpu.sync_copy(acc_sh.at[pl.ds(s * RPS, RPS)],
                        s_hbm.at[c, pl.ds(s * RPS, RPS)])

    return sc_gather, sc_scatter


def _dot(a, b):
    return jnp.dot(a, b, preferred_element_type=_f32)


def _a1_body(xn_ref, kno_ref, kn_ref, kna_ref, xn0_ref, y_ref):
    xn0 = _dot(xn_ref[...], kno_ref[...])
    xn0_ref[...] = xn0
    wg = jnp.concatenate([kn_ref[...], 0.5 * kna_ref[...]], axis=1)
    y_ref[...] = _dot(xn0, wg)


def _tc_open_node(xnT, knoT, knT, knaT):
    return pl.pallas_call(
        _a1_body,
        grid=(NPAD // BRN,),
        in_specs=[pl.BlockSpec((BRN, C), lambda i: (i, 0)),
                  pl.BlockSpec((C, C), lambda i: (0, 0)),
                  pl.BlockSpec((C, 64), lambda i: (0, 0)),
                  pl.BlockSpec((C, 64), lambda i: (0, 0))],
        out_specs=[pl.BlockSpec((BRN, C), lambda i: (i, 0)),
                   pl.BlockSpec((BRN, C), lambda i: (i, 0))],
        out_shape=[jax.ShapeDtypeStruct((NPAD, C), _f32),
                   jax.ShapeDtypeStruct((NPAD, C), _f32)],
    )(xnT, knoT, knT, knaT)


def _a2_body(xe_ref, keo_ref, out_ref):
    out_ref[...] = _dot(xe_ref[...], keo_ref[...])


def _tc_open_edge(xeT, keoT):
    return pl.pallas_call(
        _a2_body,
        grid=(EPAD // BRE,),
        in_specs=[pl.BlockSpec((BRE, 16), lambda i: (i, 0)),
                  pl.BlockSpec((16, C), lambda i: (0, 0))],
        out_specs=pl.BlockSpec((BRE, C), lambda i: (i, 0)),
        out_shape=jax.ShapeDtypeStruct((EPAD, C), _f32),
    )(xeT, keoT)


def _edge_update(gi, gj, xe):
    ch = lax.broadcasted_iota(jnp.int32, gi.shape, 1)
    a = jnp.where(ch < 64, gi - gj, gi + gj)
    a = a - jnp.mean(a, axis=1, keepdims=True)
    a = a / jnp.sqrt(jnp.sum(a * a, axis=1, keepdims=True) + 1e-3)
    return xe + 0.1 * jnp.maximum(a, 0.0)


def _e_body(gi_ref, gj_ref, xet_ref, keo_ref, out_ref):
    xe0 = _dot(xet_ref[...], keo_ref[...])
    xe_new = _edge_update(gi_ref[...], gj_ref[...], xe0)
    valid = pl.program_id(0) < (E // BRE)
    out_ref[...] = jnp.where(valid, xe_new, 0.0)


def _tc_edge_open(gi, gj, xeT, keoT):
    return pl.pallas_call(
        _e_body,
        grid=(EPAD // BRE,),
        in_specs=[pl.BlockSpec((BRE, C), lambda i: (i, 0))] * 2 +
                 [pl.BlockSpec((BRE, 16), lambda i: (i, 0)),
                  pl.BlockSpec((16, C), lambda i: (0, 0))],
        out_specs=pl.BlockSpec((BRE, C), lambda i: (i, 0)),
        out_shape=jax.ShapeDtypeStruct((EPAD, C), _f32),
    )(gi, gj, xeT, keoT)


def _ec_body(gi_ref, gj_ref, xe_ref, kec_ref, out_ref, cl_ref):
    xe_new = _edge_update(gi_ref[...], gj_ref[...], xe_ref[...])
    valid = pl.program_id(0) < (E // BRE)
    xe_new = jnp.where(valid, xe_new, 0.0)
    out_ref[...] = xe_new
    cl_ref[...] = lax.dot_general(kec_ref[...], xe_new,
                                  (((1,), (1,)), ((), ())),
                                  preferred_element_type=_f32)


def _tc_edge_close(gi, gj, xe, kec):
    return pl.pallas_call(
        _ec_body,
        grid=(EPAD // BRE,),
        in_specs=[pl.BlockSpec((BRE, C), lambda i: (i, 0))] * 3 +
                 [pl.BlockSpec((16, C), lambda i: (0, 0))],
        out_specs=[pl.BlockSpec((BRE, C), lambda i: (i, 0)),
                   pl.BlockSpec((16, BRE), lambda i: (0, i))],
        out_shape=[jax.ShapeDtypeStruct((EPAD, C), _f32),
                   jax.ShapeDtypeStruct((16, EPAD), _f32)],
    )(gi, gj, xe, kec)


def _node_new(xn_ref, si_ref, sj_ref, kei_ref, kea_ref):
    wi = 0.1 * (kei_ref[...] + 0.5 * kea_ref[...])
    wj = 0.1 * (0.5 * kea_ref[...] - kei_ref[...])
    return xn_ref[...] + _dot(si_ref[0], wi) + _dot(sj_ref[0], wj)


def _n_body(xn_ref, si_ref, sj_ref, kei_ref, kea_ref, kn_ref, kna_ref,
            xn1_ref, y_ref):
    xn1 = _node_new(xn_ref, si_ref, sj_ref, kei_ref, kea_ref)
    xn1_ref[...] = xn1
    wg = jnp.concatenate([kn_ref[...], 0.5 * kna_ref[...]], axis=1)
    y_ref[...] = _dot(xn1, wg)


def _tc_node(xn, s2, keiT, keaT, knT, knaT):
    return pl.pallas_call(
        _n_body,
        grid=(NPAD // BRN,),
        in_specs=[pl.BlockSpec((BRN, C), lambda i: (i, 0)),
                  pl.BlockSpec((1, BRN, C), lambda i: (0, i, 0)),
                  pl.BlockSpec((1, BRN, C), lambda i: (1, i, 0)),
                  pl.BlockSpec((C, C), lambda i: (0, 0)),
                  pl.BlockSpec((C, C), lambda i: (0, 0)),
                  pl.BlockSpec((C, 64), lambda i: (0, 0)),
                  pl.BlockSpec((C, 64), lambda i: (0, 0))],
        out_specs=[pl.BlockSpec((BRN, C), lambda i: (i, 0)),
                   pl.BlockSpec((BRN, C), lambda i: (i, 0))],
        out_shape=[jax.ShapeDtypeStruct((NPAD, C), _f32),
                   jax.ShapeDtypeStruct((NPAD, C), _f32)],
    )(xn, s2, s2, keiT, keaT, knT, knaT)


def _nf_body(xn_ref, si_ref, sj_ref, kei_ref, kea_ref, knc_ref, out_ref):
    xn1 = _node_new(xn_ref, si_ref, sj_ref, kei_ref, kea_ref)
    out_ref[...] = lax.dot_general(knc_ref[...], xn1,
                                   (((1,), (1,)), ((), ())),
                                   preferred_element_type=_f32)[None]


def _tc_node_final(xn, s2, keiT, keaT, knc):
    return pl.pallas_call(
        _nf_body,
        grid=(NPAD // BRN,),
        in_specs=[pl.BlockSpec((BRN, C), lambda i: (i, 0)),
                  pl.BlockSpec((1, BRN, C), lambda i: (0, i, 0)),
                  pl.BlockSpec((1, BRN, C), lambda i: (1, i, 0)),
                  pl.BlockSpec((C, C), lambda i: (0, 0)),
                  pl.BlockSpec((C, C), lambda i: (0, 0)),
                  pl.BlockSpec((C, C), lambda i: (0, 0))],
        out_specs=pl.BlockSpec((1, C, BRN), lambda i: (0, 0, i)),
        out_shape=jax.ShapeDtypeStruct((1, C, N), _f32),
    )(xn, s2, s2, keiT, keaT, knc)


def kernel(xn, xe, edge_index, KNopen, KEopen, KNclose, KEclose,
           KN, KE, KNa, KEa):
    xnT = jnp.pad(xn[0].T, ((0, NPAD - N), (0, 0)))
    xeT = jnp.pad(xe[0, :, :, 0].T, ((0, EPAD - E), (0, 0)))
    ij = jnp.pad(edge_index, ((0, 0), (0, EPAD - E)))
    ij4 = ij.reshape(2, NS, 2, EPS // 128, 64)

    _sc_gather, _sc_scatter = _sc_kernels()

    knT = [KN[l].T for l in range(2)]
    knaT = [KNa[l].T for l in range(2)]
    keT = [KE[l].T for l in range(2)]
    keaT = [KEa[l].T for l in range(2)]

    xn0, y = _tc_open_node(xnT, KNopen.T, knT[0], knaT[0])

    gi, gj = _sc_gather(y, ij)
    xe1 = _tc_edge_open(gi, gj, xeT, KEopen.T)
    s2 = _sc_scatter(xe1, ij4)
    xn1, y = _tc_node(xn0, s2, keT[0], keaT[0], knT[1], knaT[1])

    gi, gj = _sc_gather(y, ij)
    xe2, xeclT = _tc_edge_close(gi, gj, xe1, KEclose)
    s2 = _sc_scatter(xe2, ij4)
    xncl = _tc_node_final(xn1, s2, keT[1], keaT[1], KNclose)

    return (xncl, xeclT[:, :E][None, :, :, None])
